# baseline (speedup 1.0000x reference)
import functools

import numpy as np

import jax
import jax.numpy as jnp
from jax.experimental import pallas as pl
from jax.experimental.pallas import tpu as pltpu


def _bise_kernel(lhs_ref, x_ref, o_ref, rhs_ref, *, H, W, C_out, BN, CB):
    @pl.when(pl.program_id(1) == 0)
    def _build_rhs():
        xs = [x_ref[n] for n in range(BN)]
        xl = jnp.concatenate(
            sum(([x[:, :1], x[:, :-1]] for x in xs), []), axis=1)
        xm = jnp.concatenate(xs, axis=1)
        xr = jnp.concatenate(
            sum(([x[:, 1:], x[:, -1:]] for x in xs), []), axis=1)
        ones = jnp.ones((1, BN * W), jnp.float32)
        rhs_ref[...] = jnp.concatenate([xl, xm, xr, ones], axis=0)

    res = jax.lax.dot_general(
        lhs_ref[...], rhs_ref[...],
        dimension_numbers=(((1,), (0,)), ((), ())),
        preferred_element_type=jnp.float32)

    out = 0.5 * jnp.tanh(res) + 0.5

    for n in range(BN):
        for c in range(CB):
            o_ref[n, c] = out[c * H:(c + 1) * H, n * W:(n + 1) * W]


def _bise_forward(x, conv_weight, raw_bias, weight_P, activation_P):
    N, Cin, H, W = x.shape
    assert Cin == 1
    C_out = conv_weight.shape[0]

    w_norm = jax.nn.sigmoid(weight_P[:, None, None] * conv_weight[:, 0, :, :])
    half_p = 0.5 * activation_P.astype(jnp.float32)
    w2 = half_p[:, None, None] * w_norm
    b2 = half_p * (-jax.nn.softplus(raw_bias) - 0.5)
    b_col = jnp.repeat(b2, H)[:, None].astype(jnp.float32)

    d0 = np.eye(H, k=-1, dtype=np.float32)
    d0[0, 0] = 1.0
    d1 = np.eye(H, dtype=np.float32)
    d2 = np.eye(H, k=1, dtype=np.float32)
    d2[-1, -1] = 1.0
    bands = jnp.stack([jnp.asarray(d0), jnp.asarray(d1), jnp.asarray(d2)])
    blocks = jnp.einsum("cij,ikl->cjkl", w2, bands)
    lhs = jnp.transpose(blocks, (0, 2, 1, 3)).reshape(C_out * H, 3 * H)
    lhs = jnp.concatenate([lhs, b_col], axis=1).astype(jnp.float32)

    x_in = x[:, 0, :, :]

    BN = 8
    CB = C_out // 2 if C_out % 2 == 0 else C_out
    assert N % BN == 0 and C_out % CB == 0

    kernel_fn = functools.partial(
        _bise_kernel, H=H, W=W, C_out=C_out, BN=BN, CB=CB)

    out = pl.pallas_call(
        kernel_fn,
        out_shape=jax.ShapeDtypeStruct((N, C_out, H, W), jnp.float32),
        grid=(N // BN, C_out // CB),
        in_specs=[
            pl.BlockSpec((CB * H, 3 * H + 1), lambda n, ch: (ch, 0)),
            pl.BlockSpec((BN, H, W), lambda n, ch: (n, 0, 0)),
        ],
        out_specs=pl.BlockSpec((BN, CB, H, W), lambda n, ch: (n, ch, 0, 0)),
        scratch_shapes=[pltpu.VMEM((3 * H + 1, BN * W), jnp.float32)],
        compiler_params=pltpu.CompilerParams(
            dimension_semantics=("parallel", "arbitrary")),
    )(lhs, x_in)

    return out


def kernel(x, conv_weight, raw_bias, weight_P, activation_P):
    return _bise_forward(x, conv_weight, raw_bias, weight_P, activation_P)


# speedup vs baseline: 1.7159x; 1.7159x over previous
import functools

import numpy as np

import jax
import jax.numpy as jnp
from jax.experimental import pallas as pl
from jax.experimental.pallas import tpu as pltpu


def _bise_kernel(lhs_ref, x_ref, o_ref, *, H, W, C_out, BN):
    xs = [x_ref[n] for n in range(BN)]

    xl = jnp.concatenate(
        sum(([x[:, :1], x[:, :-1]] for x in xs), []), axis=1)
    xm = jnp.concatenate(xs, axis=1)
    xr = jnp.concatenate(
        sum(([x[:, 1:], x[:, -1:]] for x in xs), []), axis=1)

    ones = jnp.ones((1, xm.shape[1]), jnp.float32)
    rhs = jnp.concatenate([xl, xm, xr, ones], axis=0)

    res = jax.lax.dot_general(
        lhs_ref[...], rhs,
        dimension_numbers=(((1,), (0,)), ((), ())),
        preferred_element_type=jnp.float32)

    out = 0.5 * jnp.tanh(res) + 0.5

    for n in range(BN):
        for c in range(C_out):
            o_ref[n, c] = out[c * H:(c + 1) * H, n * W:(n + 1) * W]


def _bise_forward(x, conv_weight, raw_bias, weight_P, activation_P):
    N, Cin, H, W = x.shape
    assert Cin == 1
    C_out = conv_weight.shape[0]

    w_norm = jax.nn.sigmoid(weight_P[:, None, None] * conv_weight[:, 0, :, :])
    half_p = 0.5 * activation_P.astype(jnp.float32)
    w2 = half_p[:, None, None] * w_norm
    b2 = half_p * (-jax.nn.softplus(raw_bias) - 0.5)
    b_col = jnp.repeat(b2, H)[:, None].astype(jnp.float32)

    d0 = np.eye(H, k=-1, dtype=np.float32)
    d0[0, 0] = 1.0
    d1 = np.eye(H, dtype=np.float32)
    d2 = np.eye(H, k=1, dtype=np.float32)
    d2[-1, -1] = 1.0
    bands = jnp.stack([jnp.asarray(d0), jnp.asarray(d1), jnp.asarray(d2)])
    blocks = jnp.einsum("cij,ikl->cjkl", w2, bands)
    lhs = jnp.transpose(blocks, (0, 2, 1, 3)).reshape(C_out * H, 3 * H)
    lhs = jnp.concatenate([lhs, b_col], axis=1).astype(jnp.float32)

    x_in = x[:, 0, :, :]

    BN = 16
    assert N % BN == 0

    kernel_fn = functools.partial(_bise_kernel, H=H, W=W, C_out=C_out, BN=BN)

    out = pl.pallas_call(
        kernel_fn,
        out_shape=jax.ShapeDtypeStruct((N, C_out, H, W), jnp.float32),
        grid=(N // BN,),
        in_specs=[
            pl.BlockSpec((C_out * H, 3 * H + 1), lambda n: (0, 0)),
            pl.BlockSpec((BN, H, W), lambda n: (n, 0, 0)),
        ],
        out_specs=pl.BlockSpec((BN, C_out, H, W), lambda n: (n, 0, 0, 0)),
        compiler_params=pltpu.CompilerParams(
            dimension_semantics=("parallel",)),
    )(lhs, x_in)

    return out


def kernel(x, conv_weight, raw_bias, weight_P, activation_P):
    return _bise_forward(x, conv_weight, raw_bias, weight_P, activation_P)


# speedup vs baseline: 1.7342x; 1.0106x over previous
import functools

import numpy as np

import jax
import jax.numpy as jnp
from jax.experimental import pallas as pl
from jax.experimental.pallas import tpu as pltpu


def _bise_kernel(lhs_ref, x_ref, o_ref, *, H, W, C_out, BN):
    xs = [x_ref[n] for n in range(BN)]

    xl = jnp.concatenate(
        sum(([x[:, :1], x[:, :-1]] for x in xs), []), axis=1)
    xm = jnp.concatenate(xs, axis=1)
    xr = jnp.concatenate(
        sum(([x[:, 1:], x[:, -1:]] for x in xs), []), axis=1)

    ones = jnp.ones((1, xm.shape[1]), jnp.float32)
    rhs = jnp.concatenate([xl, xm, xr, ones], axis=0)
    rhs = rhs.astype(jnp.bfloat16)

    res = jax.lax.dot_general(
        lhs_ref[...], rhs,
        dimension_numbers=(((1,), (0,)), ((), ())),
        preferred_element_type=jnp.float32)

    out = 0.5 * jnp.tanh(res) + 0.5

    for n in range(BN):
        for c in range(C_out):
            o_ref[n, c] = out[c * H:(c + 1) * H, n * W:(n + 1) * W]


def _bise_forward(x, conv_weight, raw_bias, weight_P, activation_P):
    N, Cin, H, W = x.shape
    assert Cin == 1
    C_out = conv_weight.shape[0]

    w_norm = jax.nn.sigmoid(weight_P[:, None, None] * conv_weight[:, 0, :, :])
    half_p = 0.5 * activation_P.astype(jnp.float32)
    w2 = half_p[:, None, None] * w_norm
    b2 = half_p * (-jax.nn.softplus(raw_bias) - 0.5)
    b_col = jnp.repeat(b2, H)[:, None].astype(jnp.float32)

    d0 = np.eye(H, k=-1, dtype=np.float32)
    d0[0, 0] = 1.0
    d1 = np.eye(H, dtype=np.float32)
    d2 = np.eye(H, k=1, dtype=np.float32)
    d2[-1, -1] = 1.0
    bands = jnp.stack([jnp.asarray(d0), jnp.asarray(d1), jnp.asarray(d2)])
    blocks = jnp.einsum("cij,ikl->cjkl", w2, bands)
    lhs = jnp.transpose(blocks, (0, 2, 1, 3)).reshape(C_out * H, 3 * H)
    lhs = jnp.concatenate([lhs, b_col], axis=1).astype(jnp.bfloat16)

    x_in = x[:, 0, :, :]

    BN = 16
    assert N % BN == 0

    kernel_fn = functools.partial(_bise_kernel, H=H, W=W, C_out=C_out, BN=BN)

    out = pl.pallas_call(
        kernel_fn,
        out_shape=jax.ShapeDtypeStruct((N, C_out, H, W), jnp.float32),
        grid=(N // BN,),
        in_specs=[
            pl.BlockSpec((C_out * H, 3 * H + 1), lambda n: (0, 0)),
            pl.BlockSpec((BN, H, W), lambda n: (n, 0, 0)),
        ],
        out_specs=pl.BlockSpec((BN, C_out, H, W), lambda n: (n, 0, 0, 0)),
        compiler_params=pltpu.CompilerParams(
            dimension_semantics=("parallel",)),
    )(lhs, x_in)

    return out


def kernel(x, conv_weight, raw_bias, weight_P, activation_P):
    return _bise_forward(x, conv_weight, raw_bias, weight_P, activation_P)
